# baseline (speedup 1.0000x reference)
import functools

import jax
import jax.numpy as jnp
from jax import lax
from jax.experimental import pallas as pl
from jax.experimental.pallas import tpu as pltpu
from jax.experimental.pallas import tpu_sc as plsc

N_ATOMS = 3_200_000
N_MOL = 50_000
STD = 1.2
MEAN = -0.5

LANES = 16
NC = 2
NS = 16
NW = NC * NS
PER_W = N_ATOMS // NW
CHUNK = 10_000
NCHUNK = PER_W // CHUNK
STRIDE = CHUNK // LANES
UNROLL = 125

ACC_N = ((N_MOL + 255) // 256) * 256
SLICE = ACC_N // NS
SLICE_V = SLICE // LANES
NWAVE = 8


def _sc_body(e_hbm, i_hbm, part_hbm, acc, eb0, ib0, eb1, ib1,
             mb0, mb1, rbuf, shared, se0, si0, se1, si1, sm0, sm1):
    c = lax.axis_index("c")
    s = lax.axis_index("s")
    wid = s * NC + c
    base = wid * PER_W

    bufs = ((eb0, ib0, se0, si0), (eb1, ib1, se1, si1))

    def start(ch, which):
        eb, ib, se, si = bufs[which]
        off = base + ch * CHUNK
        pltpu.async_copy(e_hbm.at[pl.ds(off, CHUNK)], eb, se)
        pltpu.async_copy(i_hbm.at[pl.ds(off, CHUNK)], ib, si)

    def wait(which):
        eb, ib, se, si = bufs[which]
        pltpu.make_async_copy(e_hbm.at[pl.ds(0, CHUNK)], eb, se).wait()
        pltpu.make_async_copy(i_hbm.at[pl.ds(0, CHUNK)], ib, si).wait()

    start(0, 0)
    start(1, 1)

    zero = jnp.zeros((LANES,), jnp.float32)

    @plsc.parallel_loop(0, ACC_N // LANES, step=1, unroll=16)
    def _(i):
        acc[pl.ds(i * LANES, LANES)] = zero

    base_ix = lax.iota(jnp.int32, LANES) * STRIDE

    def compute(which):
        eb, ib, _, _ = bufs[which]

        @plsc.parallel_loop(0, STRIDE, step=1, unroll=UNROLL)
        def _(i):
            iv = base_ix + i
            e = plsc.load_gather(eb, [iv])
            ix = plsc.load_gather(ib, [iv])
            plsc.addupdate_scatter(acc, [ix], e * STD + MEAN)

    def pair(p, _):
        ch0 = p * 2
        wait(0)
        compute(0)

        @pl.when(ch0 + 2 < NCHUNK)
        def _():
            start(ch0 + 2, 0)

        wait(1)
        compute(1)

        @pl.when(ch0 + 3 < NCHUNK)
        def _():
            start(ch0 + 3, 1)

        return 0

    lax.fori_loop(0, NCHUNK // 2, pair, 0)

    moff = s * SLICE
    mbufs = ((mb0, sm0), (mb1, sm1))

    def mstart(t):
        mb, sm = mbufs[t % 2]
        return pltpu.async_copy(shared.at[pl.ds(t * ACC_N + moff, SLICE)], mb, sm)

    for wave in range(NS // NWAVE):
        lo_t, hi_t = wave * NWAVE, (wave + 1) * NWAVE

        @pl.when(jnp.logical_and(s >= lo_t, s < hi_t))
        def _():
            pltpu.sync_copy(acc, shared.at[pl.ds((s - lo_t) * ACC_N, ACC_N)])

        plsc.subcore_barrier()

        if wave == 0:
            pltpu.sync_copy(shared.at[pl.ds(moff, SLICE)], rbuf)
            first = 1
        else:
            first = 0
        mpend = {first % 2: mstart(first)}
        for t in range(first, NWAVE):
            mpend[t % 2].wait()
            mb, _ = mbufs[t % 2]
            if t + 1 < NWAVE:
                mpend[(t + 1) % 2] = mstart(t + 1)

            @plsc.parallel_loop(0, SLICE_V, step=1, unroll=14)
            def _(i):
                o = i * LANES
                rbuf[pl.ds(o, LANES)] = rbuf[pl.ds(o, LANES)] + mb[pl.ds(o, LANES)]

        if wave == 0:
            plsc.subcore_barrier()

    pltpu.sync_copy(rbuf, part_hbm.at[pl.ds(c * ACC_N + moff, SLICE)])


_sc_call = functools.partial(
    pl.kernel,
    out_type=jax.ShapeDtypeStruct((NC * ACC_N,), jnp.float32),
    mesh=plsc.VectorSubcoreMesh(
        core_axis_name="c", subcore_axis_name="s", num_cores=NC, num_subcores=NS
    ),
    scratch_types=[
        pltpu.VMEM((ACC_N,), jnp.float32),
        pltpu.VMEM((CHUNK,), jnp.float32),
        pltpu.VMEM((CHUNK,), jnp.int32),
        pltpu.VMEM((CHUNK,), jnp.float32),
        pltpu.VMEM((CHUNK,), jnp.int32),
        pltpu.VMEM((SLICE,), jnp.float32),
        pltpu.VMEM((SLICE,), jnp.float32),
        pltpu.VMEM((SLICE,), jnp.float32),
        pltpu.MemorySpace.VMEM_SHARED((NWAVE * ACC_N,), jnp.float32),
        pltpu.SemaphoreType.DMA,
        pltpu.SemaphoreType.DMA,
        pltpu.SemaphoreType.DMA,
        pltpu.SemaphoreType.DMA,
        pltpu.SemaphoreType.DMA,
        pltpu.SemaphoreType.DMA,
    ],
    compiler_params=pltpu.CompilerParams(needs_layout_passes=False),
)(_sc_body)


def _merge_body(p_ref, o_ref):
    o_ref[...] = p_ref[pl.ds(0, N_MOL)] + p_ref[pl.ds(ACC_N, N_MOL)]


_merge_call = pl.pallas_call(
    _merge_body,
    out_shape=jax.ShapeDtypeStruct((N_MOL,), jnp.float32),
)


_CP_BLK = 128_000


def _copy_body(e_ref, o_ref):
    o_ref[...] = e_ref[...]


_copy_call = pl.pallas_call(
    _copy_body,
    grid=(N_ATOMS // _CP_BLK,),
    in_specs=[pl.BlockSpec((_CP_BLK,), lambda i: (i,))],
    out_specs=pl.BlockSpec((_CP_BLK,), lambda i: (i,)),
    out_shape=jax.ShapeDtypeStruct((N_ATOMS,), jnp.float32),
)


def kernel(per_atom_energy, atomic_subsystem_indices):
    e = per_atom_energy.reshape(N_ATOMS)
    idx = atomic_subsystem_indices.astype(jnp.int32)
    partials = _sc_call(e, idx)
    per_molecule = _merge_call(partials)
    e_out = _copy_call(e)
    return (per_molecule, e_out.reshape(N_ATOMS, 1))


# speedup vs baseline: 1.1269x; 1.1269x over previous
import functools

import jax
import jax.numpy as jnp
from jax import lax
from jax.experimental import pallas as pl
from jax.experimental.pallas import tpu as pltpu
from jax.experimental.pallas import tpu_sc as plsc

N_ATOMS = 3_200_000
N_MOL = 50_000
STD = 1.2
MEAN = -0.5

LANES = 16
NC = 2
NS = 16
NW = NC * NS
PER_W = N_ATOMS // NW
CHUNK = 10_000
NCHUNK = PER_W // CHUNK
STRIDE = CHUNK // LANES
UNROLL = 25

ACC_N = ((N_MOL + 255) // 256) * 256
SLICE = ACC_N // NS
SLICE_V = SLICE // LANES
NWAVE = 8


def _sc_body(e_hbm, i_hbm, part_hbm, acc, eb0, ib0, eb1, ib1,
             mb0, mb1, rbuf, shared, se0, si0, se1, si1, sm0, sm1):
    c = lax.axis_index("c")
    s = lax.axis_index("s")
    wid = s * NC + c
    base = wid * PER_W

    bufs = ((eb0, ib0, se0, si0), (eb1, ib1, se1, si1))

    def start(ch, which):
        eb, ib, se, si = bufs[which]
        off = base + ch * CHUNK
        pltpu.async_copy(e_hbm.at[pl.ds(off, CHUNK)], eb, se)
        pltpu.async_copy(i_hbm.at[pl.ds(off, CHUNK)], ib, si)

    def wait(which):
        eb, ib, se, si = bufs[which]
        pltpu.make_async_copy(e_hbm.at[pl.ds(0, CHUNK)], eb, se).wait()
        pltpu.make_async_copy(i_hbm.at[pl.ds(0, CHUNK)], ib, si).wait()

    start(0, 0)
    start(1, 1)

    zero = jnp.zeros((LANES,), jnp.float32)

    @plsc.parallel_loop(0, ACC_N // LANES, step=1, unroll=16)
    def _(i):
        acc[pl.ds(i * LANES, LANES)] = zero

    base_ix = lax.iota(jnp.int32, LANES) * STRIDE

    def compute(which):
        eb, ib, _, _ = bufs[which]

        @plsc.parallel_loop(0, STRIDE, step=1, unroll=UNROLL)
        def _(i):
            iv = base_ix + i
            e = plsc.load_gather(eb, [iv])
            ix = plsc.load_gather(ib, [iv])
            plsc.addupdate_scatter(acc, [ix], e * STD + MEAN)

    def pair(p, _):
        ch0 = p * 2
        wait(0)
        compute(0)

        @pl.when(ch0 + 2 < NCHUNK)
        def _():
            start(ch0 + 2, 0)

        wait(1)
        compute(1)

        @pl.when(ch0 + 3 < NCHUNK)
        def _():
            start(ch0 + 3, 1)

        return 0

    lax.fori_loop(0, NCHUNK // 2, pair, 0)

    moff = s * SLICE
    mbufs = ((mb0, sm0), (mb1, sm1))

    def mstart(t):
        mb, sm = mbufs[t % 2]
        return pltpu.async_copy(shared.at[pl.ds(t * ACC_N + moff, SLICE)], mb, sm)

    for wave in range(NS // NWAVE):
        lo_t, hi_t = wave * NWAVE, (wave + 1) * NWAVE

        @pl.when(jnp.logical_and(s >= lo_t, s < hi_t))
        def _():
            pltpu.sync_copy(acc, shared.at[pl.ds((s - lo_t) * ACC_N, ACC_N)])

        plsc.subcore_barrier()

        if wave == 0:
            pltpu.sync_copy(shared.at[pl.ds(moff, SLICE)], rbuf)
            first = 1
        else:
            first = 0
        mpend = {first % 2: mstart(first)}
        for t in range(first, NWAVE):
            mpend[t % 2].wait()
            mb, _ = mbufs[t % 2]
            if t + 1 < NWAVE:
                mpend[(t + 1) % 2] = mstart(t + 1)

            @plsc.parallel_loop(0, SLICE_V, step=1, unroll=14)
            def _(i):
                o = i * LANES
                rbuf[pl.ds(o, LANES)] = rbuf[pl.ds(o, LANES)] + mb[pl.ds(o, LANES)]

        if wave == 0:
            plsc.subcore_barrier()

    pltpu.sync_copy(rbuf, part_hbm.at[pl.ds(c * ACC_N + moff, SLICE)])


_sc_call = functools.partial(
    pl.kernel,
    out_type=jax.ShapeDtypeStruct((NC * ACC_N,), jnp.float32),
    mesh=plsc.VectorSubcoreMesh(
        core_axis_name="c", subcore_axis_name="s", num_cores=NC, num_subcores=NS
    ),
    scratch_types=[
        pltpu.VMEM((ACC_N,), jnp.float32),
        pltpu.VMEM((CHUNK,), jnp.float32),
        pltpu.VMEM((CHUNK,), jnp.int32),
        pltpu.VMEM((CHUNK,), jnp.float32),
        pltpu.VMEM((CHUNK,), jnp.int32),
        pltpu.VMEM((SLICE,), jnp.float32),
        pltpu.VMEM((SLICE,), jnp.float32),
        pltpu.VMEM((SLICE,), jnp.float32),
        pltpu.MemorySpace.VMEM_SHARED((NWAVE * ACC_N,), jnp.float32),
        pltpu.SemaphoreType.DMA,
        pltpu.SemaphoreType.DMA,
        pltpu.SemaphoreType.DMA,
        pltpu.SemaphoreType.DMA,
        pltpu.SemaphoreType.DMA,
        pltpu.SemaphoreType.DMA,
    ],
    compiler_params=pltpu.CompilerParams(needs_layout_passes=False),
)(_sc_body)


def _merge_body(p_ref, o_ref):
    o_ref[...] = p_ref[pl.ds(0, N_MOL)] + p_ref[pl.ds(ACC_N, N_MOL)]


_merge_call = pl.pallas_call(
    _merge_body,
    out_shape=jax.ShapeDtypeStruct((N_MOL,), jnp.float32),
)


_CP_BLK = 128_000


def _copy_body(e_ref, o_ref):
    o_ref[...] = e_ref[...]


_copy_call = pl.pallas_call(
    _copy_body,
    grid=(N_ATOMS // _CP_BLK,),
    in_specs=[pl.BlockSpec((_CP_BLK,), lambda i: (i,))],
    out_specs=pl.BlockSpec((_CP_BLK,), lambda i: (i,)),
    out_shape=jax.ShapeDtypeStruct((N_ATOMS,), jnp.float32),
)


def kernel(per_atom_energy, atomic_subsystem_indices):
    e = per_atom_energy.reshape(N_ATOMS)
    idx = atomic_subsystem_indices.astype(jnp.int32)
    partials = _sc_call(e, idx)
    per_molecule = _merge_call(partials)
    e_out = _copy_call(e)
    return (per_molecule, e_out.reshape(N_ATOMS, 1))
